# baseline (speedup 1.0000x reference)
import functools

import jax
import jax.numpy as jnp
from jax.experimental import pallas as pl
from jax.experimental.pallas import tpu as pltpu


def _fused_block_kernel(x_ref, w1_ref, b1_ref, w2_ref, b2_ref, o_ref,
                        b_ref, mid_ref, *, D, H, W, C):
    M = H * W

    def kw_stack(v):
        zrow = jnp.zeros(v.shape[:-2] + (1, C), dtype=v.dtype)
        s0 = jnp.concatenate([zrow, v[..., :W - 1, :]], axis=-2)
        s2 = jnp.concatenate([v[..., 1:, :], zrow], axis=-2)
        return jnp.concatenate([s0, v, s2], axis=-1)

    def conv_rows(w_ref, epilogue):
        accs = {}

        def add(d, contrib):
            accs[d] = accs[d] + contrib if d in accs else contrib

        for dz in range(1, D + 1):
            lhs = b_ref[dz].reshape((H + 2) * W, 3 * C)
            for j0, j1 in ((0, 5), (5, 9)):
                r = jnp.dot(lhs, w_ref[:, j0 * C:j1 * C],
                            preferred_element_type=jnp.float32)
                for j in range(j0, j1):
                    kd, kh = divmod(j, 3)
                    d = dz - kd
                    if 0 <= d < D:
                        add(d, r[kh * W:kh * W + M, (j - j0) * C:(j - j0 + 1) * C])
            if dz - 2 >= 0:
                epilogue(dz - 2, accs.pop(dz - 2))
        epilogue(D - 1, accs.pop(D - 1))

    zplane_d = jnp.zeros((1, H + 2, W, 3 * C), dtype=jnp.bfloat16)
    zplane_h = jnp.zeros((D + 2, 1, W, 3 * C), dtype=jnp.bfloat16)
    b_ref[0] = zplane_d[0]
    b_ref[D + 1] = zplane_d[0]
    b_ref[:, 0] = zplane_h[:, 0]
    b_ref[:, H + 1] = zplane_h[:, 0]
    b_ref[1:D + 1, 1:H + 1, :, :] = kw_stack(x_ref[0].astype(jnp.bfloat16))

    zmid_d = jnp.zeros((H + 2, W, C), dtype=jnp.bfloat16)
    zmid_h = jnp.zeros((D + 2, W, C), dtype=jnp.bfloat16)
    mid_ref[0] = zmid_d
    mid_ref[D + 1] = zmid_d
    mid_ref[:, 0] = zmid_h
    mid_ref[:, H + 1] = zmid_h

    def epi1(d, acc):
        y = jnp.maximum(acc + b1_ref[...], 0.0).astype(jnp.bfloat16)
        mid_ref[d + 1, 1:H + 1, :, :] = y.reshape(H, W, C)

    conv_rows(w1_ref, epi1)

    b_ref[...] = kw_stack(mid_ref[...])

    def epi2(d, acc):
        res = x_ref[0, d].reshape(M, C)
        z = jnp.maximum(acc + b2_ref[...] + res, 0.0)
        o_ref[0, d] = z.reshape(H, W, C)

    conv_rows(w2_ref, epi2)


def kernel(x, w1, s1, b1, w2, s2, b2):
    N, D, H, W, C = x.shape

    w1f = (w1 * s1).astype(jnp.bfloat16).transpose(2, 3, 0, 1, 4).reshape(3 * C, 9 * C)
    w2f = (w2 * s2).astype(jnp.bfloat16).transpose(2, 3, 0, 1, 4).reshape(3 * C, 9 * C)
    b1f = b1.reshape(1, C).astype(jnp.float32)
    b2f = b2.reshape(1, C).astype(jnp.float32)

    body = functools.partial(_fused_block_kernel, D=D, H=H, W=W, C=C)

    flops = 2 * 2 * N * D * H * W * 27 * C * C + 4 * N * D * H * W * C
    bytes_accessed = (x.size * 4 + 2 * 27 * C * C * 2 + N * D * H * W * C * 4)

    return pl.pallas_call(
        body,
        out_shape=jax.ShapeDtypeStruct((N, D, H, W, C), x.dtype),
        grid=(N,),
        in_specs=[
            pl.BlockSpec((1, D, H, W, C), lambda n: (n, 0, 0, 0, 0)),
            pl.BlockSpec((3 * C, 9 * C), lambda n: (0, 0)),
            pl.BlockSpec((1, C), lambda n: (0, 0)),
            pl.BlockSpec((3 * C, 9 * C), lambda n: (0, 0)),
            pl.BlockSpec((1, C), lambda n: (0, 0)),
        ],
        out_specs=pl.BlockSpec((1, D, H, W, C), lambda n: (n, 0, 0, 0, 0)),
        scratch_shapes=[
            pltpu.VMEM((D + 2, H + 2, W, 3 * C), jnp.bfloat16),
            pltpu.VMEM((D + 2, H + 2, W, C), jnp.bfloat16),
        ],
        compiler_params=pltpu.CompilerParams(
            dimension_semantics=("parallel",),
            vmem_limit_bytes=56 * 1024 * 1024),
        cost_estimate=pl.CostEstimate(
            flops=int(flops), transcendentals=0, bytes_accessed=int(bytes_accessed)),
    )(x, w1f, b1f, w2f, b2f)


# speedup vs baseline: 2.1987x; 2.1987x over previous
import functools

import jax
import jax.numpy as jnp
from jax.experimental import pallas as pl
from jax.experimental.pallas import tpu as pltpu


def _fused_block_kernel(x_ref, w1_ref, b1_ref, w2_ref, b2_ref, o_ref,
                        b_ref, mid_ref, *, D, H, W, C):
    M = H * W

    def kw_stack(v):
        zrow = jnp.zeros(v.shape[:-2] + (1, C), dtype=v.dtype)
        s0 = jnp.concatenate([zrow, v[..., :W - 1, :]], axis=-2)
        s2 = jnp.concatenate([v[..., 1:, :], zrow], axis=-2)
        return jnp.concatenate([s0, v, s2], axis=-1)

    def conv_rows(w_ref, epilogue):
        accs = {}

        def add(d, contrib):
            accs[d] = accs[d] + contrib if d in accs else contrib

        for dz in range(1, D + 1):
            lhs = b_ref[dz].reshape((H + 2) * W, 3 * C)
            r = jnp.dot(lhs, w_ref[...], preferred_element_type=jnp.float32)
            for kd in range(3):
                d = dz - kd
                if 0 <= d < D:
                    for kh in range(3):
                        j = kd * 3 + kh
                        add(d, r[kh * W:kh * W + M, j * C:(j + 1) * C])
            if dz - 2 >= 0:
                epilogue(dz - 2, accs.pop(dz - 2))
        epilogue(D - 1, accs.pop(D - 1))

    zplane_d = jnp.zeros((1, H + 2, W, 3 * C), dtype=jnp.bfloat16)
    zplane_h = jnp.zeros((D + 2, 1, W, 3 * C), dtype=jnp.bfloat16)
    b_ref[0] = zplane_d[0]
    b_ref[D + 1] = zplane_d[0]
    b_ref[:, 0] = zplane_h[:, 0]
    b_ref[:, H + 1] = zplane_h[:, 0]
    b_ref[1:D + 1, 1:H + 1, :, :] = kw_stack(x_ref[0].astype(jnp.bfloat16))

    zmid_d = jnp.zeros((H + 2, W, C), dtype=jnp.bfloat16)
    zmid_h = jnp.zeros((D + 2, W, C), dtype=jnp.bfloat16)
    mid_ref[0] = zmid_d
    mid_ref[D + 1] = zmid_d
    mid_ref[:, 0] = zmid_h
    mid_ref[:, H + 1] = zmid_h

    def epi1(d, acc):
        y = jnp.maximum(acc + b1_ref[...], 0.0).astype(jnp.bfloat16)
        mid_ref[d + 1, 1:H + 1, :, :] = y.reshape(H, W, C)

    conv_rows(w1_ref, epi1)

    b_ref[...] = kw_stack(mid_ref[...])

    def epi2(d, acc):
        res = x_ref[0, d].reshape(M, C)
        z = jnp.maximum(acc + b2_ref[...] + res, 0.0)
        o_ref[0, d] = z.reshape(H, W, C)

    conv_rows(w2_ref, epi2)


def kernel(x, w1, s1, b1, w2, s2, b2):
    N, D, H, W, C = x.shape

    w1f = (w1 * s1).astype(jnp.bfloat16).transpose(2, 3, 0, 1, 4).reshape(3 * C, 9 * C)
    w2f = (w2 * s2).astype(jnp.bfloat16).transpose(2, 3, 0, 1, 4).reshape(3 * C, 9 * C)
    b1f = b1.reshape(1, C).astype(jnp.float32)
    b2f = b2.reshape(1, C).astype(jnp.float32)

    body = functools.partial(_fused_block_kernel, D=D, H=H, W=W, C=C)

    flops = 2 * 2 * N * D * H * W * 27 * C * C + 4 * N * D * H * W * C
    bytes_accessed = (x.size * 4 + 2 * 27 * C * C * 2 + N * D * H * W * C * 4)

    return pl.pallas_call(
        body,
        out_shape=jax.ShapeDtypeStruct((N, D, H, W, C), x.dtype),
        grid=(N,),
        in_specs=[
            pl.BlockSpec((1, D, H, W, C), lambda n: (n, 0, 0, 0, 0)),
            pl.BlockSpec((3 * C, 9 * C), lambda n: (0, 0)),
            pl.BlockSpec((1, C), lambda n: (0, 0)),
            pl.BlockSpec((3 * C, 9 * C), lambda n: (0, 0)),
            pl.BlockSpec((1, C), lambda n: (0, 0)),
        ],
        out_specs=pl.BlockSpec((1, D, H, W, C), lambda n: (n, 0, 0, 0, 0)),
        scratch_shapes=[
            pltpu.VMEM((D + 2, H + 2, W, 3 * C), jnp.bfloat16),
            pltpu.VMEM((D + 2, H + 2, W, C), jnp.bfloat16),
        ],
        compiler_params=pltpu.CompilerParams(
            dimension_semantics=("parallel",),
            vmem_limit_bytes=56 * 1024 * 1024),
        cost_estimate=pl.CostEstimate(
            flops=int(flops), transcendentals=0, bytes_accessed=int(bytes_accessed)),
    )(x, w1f, b1f, w2f, b2f)


# speedup vs baseline: 2.2577x; 1.0268x over previous
import functools

import jax
import jax.numpy as jnp
from jax.experimental import pallas as pl
from jax.experimental.pallas import tpu as pltpu


def _fused_block_kernel(x_ref, w1_ref, b1_ref, w2_ref, b2_ref, o_ref,
                        b_ref, c_ref, *, D, H, W, C):
    M = H * W

    def kw_stack(v):
        zrow = jnp.zeros(v.shape[:-2] + (1, C), dtype=v.dtype)
        s0 = jnp.concatenate([zrow, v[..., :W - 1, :]], axis=-2)
        s2 = jnp.concatenate([v[..., 1:, :], zrow], axis=-2)
        return jnp.concatenate([s0, v, s2], axis=-1)

    def zero_halo(ref):
        zd = jnp.zeros((H + 2, W, 3 * C), dtype=jnp.bfloat16)
        zh = jnp.zeros((D + 2, W, 3 * C), dtype=jnp.bfloat16)
        ref[0] = zd
        ref[D + 1] = zd
        ref[:, 0] = zh
        ref[:, H + 1] = zh

    def conv_rows(src_ref, w_ref, epilogue):
        T = jnp.zeros((M, 3 * C), dtype=jnp.float32)
        for dz in range(1, D + 1):
            lhs = src_ref[dz].reshape((H + 2) * W, 3 * C)
            r = jnp.dot(lhs, w_ref[...], preferred_element_type=jnp.float32)
            T = jnp.concatenate(
                [jnp.zeros((M, C), dtype=jnp.float32), T[:, :2 * C]], axis=-1)
            for kh in range(3):
                T = T + r[kh * W:kh * W + M, kh * 3 * C:(kh + 1) * 3 * C]
            if dz >= 2:
                epilogue(dz - 2, T[:, 2 * C:])
        epilogue(D - 1, T[:, C:2 * C])

    zero_halo(b_ref)
    b_ref[1:D + 1, 1:H + 1, :, :] = kw_stack(x_ref[0].astype(jnp.bfloat16))
    zero_halo(c_ref)

    def epi1(d, acc):
        y = jnp.maximum(acc + b1_ref[...], 0.0).astype(jnp.bfloat16)
        c_ref[d + 1, 1:H + 1, :, :] = kw_stack(y.reshape(H, W, C))

    conv_rows(b_ref, w1_ref, epi1)

    def epi2(d, acc):
        res = x_ref[0, d].reshape(M, C)
        z = jnp.maximum(acc + b2_ref[...] + res, 0.0)
        o_ref[0, d] = z.reshape(H, W, C)

    conv_rows(c_ref, w2_ref, epi2)


def kernel(x, w1, s1, b1, w2, s2, b2):
    N, D, H, W, C = x.shape

    w1f = (w1 * s1).astype(jnp.bfloat16).transpose(2, 3, 1, 0, 4).reshape(3 * C, 9 * C)
    w2f = (w2 * s2).astype(jnp.bfloat16).transpose(2, 3, 1, 0, 4).reshape(3 * C, 9 * C)
    b1f = b1.reshape(1, C).astype(jnp.float32)
    b2f = b2.reshape(1, C).astype(jnp.float32)

    body = functools.partial(_fused_block_kernel, D=D, H=H, W=W, C=C)

    flops = 2 * 2 * N * D * H * W * 27 * C * C + 4 * N * D * H * W * C
    bytes_accessed = (x.size * 4 + 2 * 27 * C * C * 2 + N * D * H * W * C * 4)

    return pl.pallas_call(
        body,
        out_shape=jax.ShapeDtypeStruct((N, D, H, W, C), x.dtype),
        grid=(N,),
        in_specs=[
            pl.BlockSpec((1, D, H, W, C), lambda n: (n, 0, 0, 0, 0)),
            pl.BlockSpec((3 * C, 9 * C), lambda n: (0, 0)),
            pl.BlockSpec((1, C), lambda n: (0, 0)),
            pl.BlockSpec((3 * C, 9 * C), lambda n: (0, 0)),
            pl.BlockSpec((1, C), lambda n: (0, 0)),
        ],
        out_specs=pl.BlockSpec((1, D, H, W, C), lambda n: (n, 0, 0, 0, 0)),
        scratch_shapes=[
            pltpu.VMEM((D + 2, H + 2, W, 3 * C), jnp.bfloat16),
            pltpu.VMEM((D + 2, H + 2, W, 3 * C), jnp.bfloat16),
        ],
        compiler_params=pltpu.CompilerParams(
            dimension_semantics=("parallel",),
            vmem_limit_bytes=56 * 1024 * 1024),
        cost_estimate=pl.CostEstimate(
            flops=int(flops), transcendentals=0, bytes_accessed=int(bytes_accessed)),
    )(x, w1f, b1f, w2f, b2f)


# speedup vs baseline: 3.0335x; 1.3436x over previous
import functools

import jax
import jax.numpy as jnp
from jax.experimental import pallas as pl
from jax.experimental.pallas import tpu as pltpu


def _fused_block_kernel(x_ref, w1_ref, b1_ref, w2_ref, b2_ref, o_ref,
                        b_ref, c_ref, *, D, H, W, C):
    M = H * W

    def kw_stack(v):
        zrow = jnp.zeros(v.shape[:-2] + (1, C), dtype=v.dtype)
        s0 = jnp.concatenate([zrow, v[..., :W - 1, :]], axis=-2)
        s2 = jnp.concatenate([v[..., 1:, :], zrow], axis=-2)
        return jnp.concatenate([s0, v, s2], axis=-1)

    def zero_halo(ref):
        zd = jnp.zeros((H + 2, W, 3 * C), dtype=jnp.bfloat16)
        zh = jnp.zeros((D + 2, W, 3 * C), dtype=jnp.bfloat16)
        ref[0] = zd
        ref[D + 1] = zd
        ref[:, 0] = zh
        ref[:, H + 1] = zh

    def conv_rows(src_ref, w_ref, epilogue):
        T = jnp.zeros((M, 3 * C), dtype=jnp.float32)
        for dz in range(1, D + 1):
            lhs = src_ref[dz].reshape((H + 2) * W, 3 * C)
            r = jnp.dot(lhs, w_ref[...], preferred_element_type=jnp.float32)
            T = jnp.concatenate(
                [jnp.zeros((M, C), dtype=jnp.float32), T[:, :2 * C]], axis=-1)
            for kh in range(3):
                T = T + r[kh * W:kh * W + M, kh * 4 * C:kh * 4 * C + 3 * C]
            if dz >= 2:
                epilogue(dz - 2, T[:, 2 * C:])
        epilogue(D - 1, T[:, C:2 * C])

    zero_halo(b_ref)
    b_ref[1:D + 1, 1:H + 1, :, :] = kw_stack(x_ref[0].astype(jnp.bfloat16))
    zero_halo(c_ref)

    def epi1(d, acc):
        y = jnp.maximum(acc + b1_ref[...], 0.0).astype(jnp.bfloat16)
        c_ref[d + 1, 1:H + 1, :, :] = kw_stack(y.reshape(H, W, C))

    conv_rows(b_ref, w1_ref, epi1)

    def epi2(d, acc):
        res = x_ref[0, d].reshape(M, C)
        z = jnp.maximum(acc + b2_ref[...] + res, 0.0)
        o_ref[0, d] = z.reshape(H, W, C)

    conv_rows(c_ref, w2_ref, epi2)


def kernel(x, w1, s1, b1, w2, s2, b2):
    N, D, H, W, C = x.shape

    def wfold(w, s):
        wt = (w * s).astype(jnp.bfloat16).transpose(2, 3, 1, 0, 4)
        wt = wt.reshape(3 * C, 3, 3 * C)
        return jnp.pad(wt, ((0, 0), (0, 0), (0, C))).reshape(3 * C, 12 * C)

    w1f = wfold(w1, s1)
    w2f = wfold(w2, s2)
    b1f = b1.reshape(1, C).astype(jnp.float32)
    b2f = b2.reshape(1, C).astype(jnp.float32)

    body = functools.partial(_fused_block_kernel, D=D, H=H, W=W, C=C)

    flops = 2 * 2 * N * D * H * W * 27 * C * C + 4 * N * D * H * W * C
    bytes_accessed = (x.size * 4 + 2 * 27 * C * C * 2 + N * D * H * W * C * 4)

    return pl.pallas_call(
        body,
        out_shape=jax.ShapeDtypeStruct((N, D, H, W, C), x.dtype),
        grid=(N,),
        in_specs=[
            pl.BlockSpec((1, D, H, W, C), lambda n: (n, 0, 0, 0, 0)),
            pl.BlockSpec((3 * C, 12 * C), lambda n: (0, 0)),
            pl.BlockSpec((1, C), lambda n: (0, 0)),
            pl.BlockSpec((3 * C, 12 * C), lambda n: (0, 0)),
            pl.BlockSpec((1, C), lambda n: (0, 0)),
        ],
        out_specs=pl.BlockSpec((1, D, H, W, C), lambda n: (n, 0, 0, 0, 0)),
        scratch_shapes=[
            pltpu.VMEM((D + 2, H + 2, W, 3 * C), jnp.bfloat16),
            pltpu.VMEM((D + 2, H + 2, W, 3 * C), jnp.bfloat16),
        ],
        compiler_params=pltpu.CompilerParams(
            dimension_semantics=("parallel",),
            vmem_limit_bytes=56 * 1024 * 1024),
        cost_estimate=pl.CostEstimate(
            flops=int(flops), transcendentals=0, bytes_accessed=int(bytes_accessed)),
    )(x, w1f, b1f, w2f, b2f)


# speedup vs baseline: 3.0569x; 1.0077x over previous
import functools

import jax
import jax.numpy as jnp
from jax.experimental import pallas as pl
from jax.experimental.pallas import tpu as pltpu


def _fused_block_kernel(x_ref, w1_ref, b1_ref, w2_ref, b2_ref, o_ref,
                        b_ref, c_ref, *, D, H, W, C):
    M = H * W

    def kw_stack(v):
        zrow = jnp.zeros(v.shape[:-2] + (1, C), dtype=v.dtype)
        s0 = jnp.concatenate([zrow, v[..., :W - 1, :]], axis=-2)
        s2 = jnp.concatenate([v[..., 1:, :], zrow], axis=-2)
        zpad = jnp.zeros(v.shape, dtype=v.dtype)
        return jnp.concatenate([s0, v, s2, zpad], axis=-1)

    def zero_halo(ref):
        zd = jnp.zeros((H + 2, W, 4 * C), dtype=jnp.bfloat16)
        zh = jnp.zeros((D + 2, W, 4 * C), dtype=jnp.bfloat16)
        ref[0] = zd
        ref[D + 1] = zd
        ref[:, 0] = zh
        ref[:, H + 1] = zh

    def conv_rows(src_ref, w_ref, epilogue):
        T = jnp.zeros((M, 3 * C), dtype=jnp.float32)
        for dz in range(1, D + 1):
            lhs = src_ref[dz].reshape((H + 2) * W, 4 * C)
            r = jnp.dot(lhs, w_ref[...], preferred_element_type=jnp.float32)
            T = jnp.concatenate(
                [jnp.zeros((M, C), dtype=jnp.float32), T[:, :2 * C]], axis=-1)
            for kh in range(3):
                T = T + r[kh * W:kh * W + M, kh * 4 * C:kh * 4 * C + 3 * C]
            if dz >= 2:
                epilogue(dz - 2, T[:, 2 * C:])
        epilogue(D - 1, T[:, C:2 * C])

    zero_halo(b_ref)
    b_ref[1:D + 1, 1:H + 1, :, :] = kw_stack(x_ref[0].astype(jnp.bfloat16))
    zero_halo(c_ref)

    def epi1(d, acc):
        y = jnp.maximum(acc + b1_ref[...], 0.0).astype(jnp.bfloat16)
        c_ref[d + 1, 1:H + 1, :, :] = kw_stack(y.reshape(H, W, C))

    conv_rows(b_ref, w1_ref, epi1)

    def epi2(d, acc):
        res = x_ref[0, d].reshape(M, C)
        z = jnp.maximum(acc + b2_ref[...] + res, 0.0)
        o_ref[0, d] = z.reshape(H, W, C)

    conv_rows(c_ref, w2_ref, epi2)


def kernel(x, w1, s1, b1, w2, s2, b2):
    N, D, H, W, C = x.shape

    def wfold(w, s):
        wt = (w * s).astype(jnp.bfloat16).transpose(2, 3, 1, 0, 4)
        wt = wt.reshape(3 * C, 3, 3 * C)
        return jnp.pad(wt, ((0, C), (0, 0), (0, C))).reshape(4 * C, 12 * C)

    w1f = wfold(w1, s1)
    w2f = wfold(w2, s2)
    b1f = b1.reshape(1, C).astype(jnp.float32)
    b2f = b2.reshape(1, C).astype(jnp.float32)

    body = functools.partial(_fused_block_kernel, D=D, H=H, W=W, C=C)

    flops = 2 * 2 * N * D * H * W * 27 * C * C + 4 * N * D * H * W * C
    bytes_accessed = (x.size * 4 + 2 * 27 * C * C * 2 + N * D * H * W * C * 4)

    return pl.pallas_call(
        body,
        out_shape=jax.ShapeDtypeStruct((N, D, H, W, C), x.dtype),
        grid=(N,),
        in_specs=[
            pl.BlockSpec((1, D, H, W, C), lambda n: (n, 0, 0, 0, 0)),
            pl.BlockSpec((4 * C, 12 * C), lambda n: (0, 0)),
            pl.BlockSpec((1, C), lambda n: (0, 0)),
            pl.BlockSpec((4 * C, 12 * C), lambda n: (0, 0)),
            pl.BlockSpec((1, C), lambda n: (0, 0)),
        ],
        out_specs=pl.BlockSpec((1, D, H, W, C), lambda n: (n, 0, 0, 0, 0)),
        scratch_shapes=[
            pltpu.VMEM((D + 2, H + 2, W, 4 * C), jnp.bfloat16),
            pltpu.VMEM((D + 2, H + 2, W, 4 * C), jnp.bfloat16),
        ],
        compiler_params=pltpu.CompilerParams(
            dimension_semantics=("parallel",),
            vmem_limit_bytes=56 * 1024 * 1024),
        cost_estimate=pl.CostEstimate(
            flops=int(flops), transcendentals=0, bytes_accessed=int(bytes_accessed)),
    )(x, w1f, b1f, w2f, b2f)
